# baseline (speedup 1.0000x reference)
import functools

import jax
import jax.numpy as jnp
from jax import lax
from jax.experimental import pallas as pl
from jax.experimental.pallas import tpu as pltpu
from jax.experimental.pallas import tpu_sc as plsc

_L = 16
_CB = 256


@functools.lru_cache(maxsize=None)
def _make_transpose(V, D):
    info = plsc.get_sparse_core_info()
    nc, ns = info.num_cores, info.num_subcores
    nw = nc * ns
    n_blocks = (V // _CB)
    v_main = n_blocks * _CB
    tail_rows = (V - v_main) * D // (2 * D)
    mesh = plsc.VectorSubcoreMesh(core_axis_name="c", subcore_axis_name="s")

    @functools.partial(
        pl.kernel,
        mesh=mesh,
        out_type=jax.ShapeDtypeStruct((V // 2, 2 * D), jnp.float32),
        scratch_types=[
            pltpu.VMEM((2, D, _CB), jnp.float32),
            pltpu.VMEM((2, _CB // 2, 2 * D), jnp.float32),
            pltpu.VMEM((tail_rows, 2 * D), jnp.float32),
            pltpu.SemaphoreType.DMA((2,)),
            pltpu.SemaphoreType.DMA((2,)),
        ],
        compiler_params=pltpu.CompilerParams(use_tc_tiling_on_sc=True, needs_layout_passes=False),
    )
    def transpose_kernel(tT_hbm, tail_hbm, scr_hbm, in_v, out_v, tail_v,
                         isem, osem):
        wid = lax.axis_index("s") * nc + lax.axis_index("c")
        nsteps = pl.cdiv(n_blocks - wid, nw)
        iota = lax.iota(jnp.int32, _L)

        def issue_read(b, slot):
            pltpu.async_copy(
                tT_hbm.at[:, pl.ds(b * _CB, _CB)], in_v.at[slot], isem.at[slot]
            )

        issue_read(wid, 0)

        def body(step, _):
            b = wid + nw * step
            slot = lax.rem(step, 2)
            nslot = lax.rem(step + 1, 2)

            @pl.when(step + 1 < nsteps)
            def _():
                issue_read(b + nw, nslot)

            pltpu.make_async_copy(
                tT_hbm.at[:, pl.ds(b * _CB, _CB)], in_v.at[slot], isem.at[slot]
            ).wait()

            @pl.when(step >= 2)
            def _():
                pltpu.make_async_copy(
                    out_v.at[slot],
                    scr_hbm.at[pl.ds(0, _CB // 2)],
                    osem.at[slot],
                ).wait()

            def row_body(jj, _):
                for q in range(2 * D // _L):
                    c0 = q * _L
                    h = c0 // D
                    c_vec = iota + (c0 - h * D)
                    r_vec = jnp.full((_L,), 0, jnp.int32) + (2 * jj + h)
                    vals = plsc.load_gather(in_v.at[slot], [c_vec, r_vec])
                    out_v[slot, jj, pl.ds(c0, _L)] = vals
                return 0

            lax.fori_loop(0, _CB // 2, row_body, 0)
            pltpu.async_copy(
                out_v.at[slot],
                scr_hbm.at[pl.ds(b * (_CB // 2), _CB // 2)],
                osem.at[slot],
            )
            return 0

        lax.fori_loop(0, nsteps, body, 0)

        @pl.when(wid == 0)
        def _():
            pltpu.sync_copy(tail_hbm, tail_v)
            pltpu.sync_copy(tail_v, scr_hbm.at[pl.ds(v_main // 2, tail_rows)])

        for slot in range(2):
            pltpu.make_async_copy(
                out_v.at[slot], scr_hbm.at[pl.ds(0, _CB // 2)], osem.at[slot]
            ).wait()

    return transpose_kernel


@functools.lru_cache(maxsize=None)
def _make_gather(J, I, D):
    info = plsc.get_sparse_core_info()
    nc, ns = info.num_cores, info.num_subcores
    nw = nc * ns
    IB = 128
    n_slabs = J * (I // IB)
    per_w = n_slabs // nw
    NBUF = 4
    assert n_slabs % nw == 0 and per_w % NBUF == 0
    mesh = plsc.VectorSubcoreMesh(core_axis_name="c", subcore_axis_name="s")

    @functools.partial(
        pl.kernel,
        mesh=mesh,
        out_type=jax.ShapeDtypeStruct((J, D, I), jnp.float32),
        scratch_types=[
            pltpu.VMEM((NBUF, IB), jnp.int32),
            pltpu.VMEM((NBUF, IB), jnp.int32),
            pltpu.VMEM((NBUF, IB, 2 * D), jnp.float32),
            pltpu.VMEM((NBUF, D, IB), jnp.float32),
            pltpu.SemaphoreType.DMA((NBUF,)),
            pltpu.SemaphoreType.DMA((NBUF,)),
            pltpu.SemaphoreType.DMA((NBUF,)),
        ],
        compiler_params=pltpu.CompilerParams(use_tc_tiling_on_sc=True, needs_layout_passes=False),
    )
    def gather_kernel(idxT_hbm, scr_hbm, out_hbm, idx_v, pair_v, gath_v,
                      slab_v, xsem, gsem, wsem):
        wid = lax.axis_index("s") * nc + lax.axis_index("c")
        iota = lax.iota(jnp.int32, _L)

        def slab_ji(s):
            g = wid * per_w + s
            return g // (I // IB), lax.rem(g, I // IB) * IB

        def issue_idx(s, slot):
            j, i0 = slab_ji(s)
            pltpu.async_copy(
                idxT_hbm.at[j, pl.ds(i0, IB)], idx_v.at[slot], xsem.at[slot]
            )

        def wait_idx(s, slot):
            j, i0 = slab_ji(s)
            pltpu.make_async_copy(
                idxT_hbm.at[j, pl.ds(i0, IB)], idx_v.at[slot], xsem.at[slot]
            ).wait()

        def issue_gather(slot):
            pltpu.async_copy(
                scr_hbm.at[pair_v.at[slot]], gath_v.at[slot], gsem.at[slot]
            )

        def wait_gather(slot):
            pltpu.make_async_copy(
                scr_hbm.at[pair_v.at[slot]], gath_v.at[slot], gsem.at[slot]
            ).wait()

        def issue_write(s, slot):
            j, i0 = slab_ji(s)
            pltpu.async_copy(
                slab_v.at[slot], out_hbm.at[j, :, pl.ds(i0, IB)], wsem.at[slot]
            )

        def wait_write(slot):
            pltpu.make_async_copy(
                slab_v.at[slot], out_hbm.at[0, :, pl.ds(0, IB)], wsem.at[slot]
            ).wait()

        def pairs_from_idx(slot):
            for q in range(IB // _L):
                v = idx_v[slot, pl.ds(q * _L, _L)]
                pair_v[slot, pl.ds(q * _L, _L)] = lax.shift_right_logical(v, 1)

        for s0 in range(3):
            issue_idx(s0, s0)
        for s0 in range(2):
            wait_idx(s0, s0)
            pairs_from_idx(s0)
            issue_gather(s0)

        def body(s, _):
            slot = lax.rem(s, NBUF)

            @pl.when(s + 3 < per_w)
            def _():
                issue_idx(s + 3, lax.rem(s + 3, NBUF))

            @pl.when(s + 2 < per_w)
            def _():
                pslot = lax.rem(s + 2, NBUF)
                wait_idx(s + 2, pslot)
                pairs_from_idx(pslot)

                @pl.when(s + 2 >= NBUF)
                def _():
                    wait_write(pslot)

                issue_gather(pslot)

            wait_gather(slot)

            def col_body(q, _):
                ii0 = q * _L
                ii_vec = iota + ii0
                h_vec = lax.bitwise_and(idx_v[slot, pl.ds(ii0, _L)], 1)
                base_col = h_vec * D
                for c in range(D):
                    vals = plsc.load_gather(
                        gath_v.at[slot], [ii_vec, base_col + c]
                    )
                    slab_v[slot, c, pl.ds(ii0, _L)] = vals
                return 0

            lax.fori_loop(0, IB // _L, col_body, 0)
            issue_write(s, slot)
            return 0

        lax.fori_loop(0, per_w, body, 0)
        for slot in range(NBUF):
            wait_write(slot)

    return gather_kernel


def kernel(x, table):
    V, D = table.shape
    n, m = x.shape
    tT = table.T
    idxT = x.T
    v_main = (V // _CB) * _CB
    tail = table[v_main:].reshape((V - v_main) // 2, 2 * D)
    scr = _make_transpose(V, D)(tT, tail)
    outT = _make_gather(m, n, D)(idxT, scr)
    return outT.transpose(2, 0, 1)


# speedup vs baseline: 2.8678x; 2.8678x over previous
import functools

import jax
import jax.numpy as jnp
from jax import lax
from jax.experimental import pallas as pl
from jax.experimental.pallas import tpu as pltpu
from jax.experimental.pallas import tpu_sc as plsc

_L = 16
_CB = 256


@functools.lru_cache(maxsize=None)
def _make_transpose(V, D):
    info = plsc.get_sparse_core_info()
    nc, ns = info.num_cores, info.num_subcores
    nw = nc * ns
    n_blocks = (V // _CB)
    v_main = n_blocks * _CB
    tail_rows = (V - v_main) * D // (2 * D)
    mesh = plsc.VectorSubcoreMesh(core_axis_name="c", subcore_axis_name="s")

    @functools.partial(
        pl.kernel,
        mesh=mesh,
        out_type=jax.ShapeDtypeStruct((V // 2, 2 * D), jnp.float32),
        scratch_types=[
            pltpu.VMEM((2, D, _CB), jnp.float32),
            pltpu.VMEM((2, _CB // 2, 2 * D), jnp.float32),
            pltpu.VMEM((tail_rows, 2 * D), jnp.float32),
            pltpu.SemaphoreType.DMA((2,)),
            pltpu.SemaphoreType.DMA((2,)),
        ],
        compiler_params=pltpu.CompilerParams(use_tc_tiling_on_sc=True, needs_layout_passes=False),
    )
    def transpose_kernel(tT_hbm, tail_hbm, scr_hbm, in_v, out_v, tail_v,
                         isem, osem):
        wid = lax.axis_index("s") * nc + lax.axis_index("c")
        nsteps = pl.cdiv(n_blocks - wid, nw)
        iota = lax.iota(jnp.int32, _L)
        cdiag = [lax.bitwise_and(iota + d, _L - 1) for d in range(_L)]

        def issue_read(b, slot):
            pltpu.async_copy(
                tT_hbm.at[:, pl.ds(b * _CB, _CB)], in_v.at[slot], isem.at[slot]
            )

        issue_read(wid, 0)

        def body(step, _):
            b = wid + nw * step
            slot = lax.rem(step, 2)
            nslot = lax.rem(step + 1, 2)

            @pl.when(step + 1 < nsteps)
            def _():
                issue_read(b + nw, nslot)

            pltpu.make_async_copy(
                tT_hbm.at[:, pl.ds(b * _CB, _CB)], in_v.at[slot], isem.at[slot]
            ).wait()

            @pl.when(step >= 2)
            def _():
                pltpu.make_async_copy(
                    out_v.at[slot],
                    scr_hbm.at[pl.ds(0, _CB // 2)],
                    osem.at[slot],
                ).wait()

            def row_body(pt, _):
                p_vec = iota + pt * _L
                r_half = [2 * p_vec, 2 * p_vec + 1]
                for q0 in range(0, 2 * D, _L):
                    h = q0 // D
                    for d in range(_L):
                        c_vec = (q0 - h * D) + cdiag[d]
                        vals = plsc.load_gather(
                            in_v.at[slot], [c_vec, r_half[h]]
                        )
                        plsc.store_scatter(
                            out_v.at[slot], [p_vec, q0 + cdiag[d]], vals
                        )
                return 0

            lax.fori_loop(0, _CB // 2 // _L, row_body, 0)
            pltpu.async_copy(
                out_v.at[slot],
                scr_hbm.at[pl.ds(b * (_CB // 2), _CB // 2)],
                osem.at[slot],
            )
            return 0

        lax.fori_loop(0, nsteps, body, 0)

        @pl.when(wid == 0)
        def _():
            pltpu.sync_copy(tail_hbm, tail_v)
            pltpu.sync_copy(tail_v, scr_hbm.at[pl.ds(v_main // 2, tail_rows)])

        for slot in range(2):
            pltpu.make_async_copy(
                out_v.at[slot], scr_hbm.at[pl.ds(0, _CB // 2)], osem.at[slot]
            ).wait()

    return transpose_kernel


@functools.lru_cache(maxsize=None)
def _make_gather(J, I, D):
    info = plsc.get_sparse_core_info()
    nc, ns = info.num_cores, info.num_subcores
    nw = nc * ns
    IB = 128
    n_slabs = J * (I // IB)
    per_w = n_slabs // nw
    NBUF = 4
    assert n_slabs % nw == 0 and per_w % NBUF == 0
    mesh = plsc.VectorSubcoreMesh(core_axis_name="c", subcore_axis_name="s")

    @functools.partial(
        pl.kernel,
        mesh=mesh,
        out_type=jax.ShapeDtypeStruct((J, D, I), jnp.float32),
        scratch_types=[
            pltpu.VMEM((NBUF, IB), jnp.int32),
            pltpu.VMEM((NBUF, IB), jnp.int32),
            pltpu.VMEM((NBUF, IB, 2 * D), jnp.float32),
            pltpu.VMEM((NBUF, D, IB), jnp.float32),
            pltpu.SemaphoreType.DMA((NBUF,)),
            pltpu.SemaphoreType.DMA((NBUF,)),
            pltpu.SemaphoreType.DMA((NBUF,)),
        ],
        compiler_params=pltpu.CompilerParams(use_tc_tiling_on_sc=True, needs_layout_passes=False),
    )
    def gather_kernel(idxT_hbm, scr_hbm, out_hbm, idx_v, pair_v, gath_v,
                      slab_v, xsem, gsem, wsem):
        wid = lax.axis_index("s") * nc + lax.axis_index("c")
        iota = lax.iota(jnp.int32, _L)
        cdiag = [lax.bitwise_and(iota + d, _L - 1) for d in range(_L)]

        def slab_ji(s):
            g = wid * per_w + s
            return g // (I // IB), lax.rem(g, I // IB) * IB

        def issue_idx(s, slot):
            j, i0 = slab_ji(s)
            pltpu.async_copy(
                idxT_hbm.at[j, pl.ds(i0, IB)], idx_v.at[slot], xsem.at[slot]
            )

        def wait_idx(s, slot):
            j, i0 = slab_ji(s)
            pltpu.make_async_copy(
                idxT_hbm.at[j, pl.ds(i0, IB)], idx_v.at[slot], xsem.at[slot]
            ).wait()

        def issue_gather(slot):
            pltpu.async_copy(
                scr_hbm.at[pair_v.at[slot]], gath_v.at[slot], gsem.at[slot]
            )

        def wait_gather(slot):
            pltpu.make_async_copy(
                scr_hbm.at[pair_v.at[slot]], gath_v.at[slot], gsem.at[slot]
            ).wait()

        def issue_write(s, slot):
            j, i0 = slab_ji(s)
            pltpu.async_copy(
                slab_v.at[slot], out_hbm.at[j, :, pl.ds(i0, IB)], wsem.at[slot]
            )

        def wait_write(slot):
            pltpu.make_async_copy(
                slab_v.at[slot], out_hbm.at[0, :, pl.ds(0, IB)], wsem.at[slot]
            ).wait()

        def pairs_from_idx(slot):
            for q in range(IB // _L):
                v = idx_v[slot, pl.ds(q * _L, _L)]
                pair_v[slot, pl.ds(q * _L, _L)] = lax.shift_right_logical(v, 1)

        for s0 in range(3):
            issue_idx(s0, s0)
        for s0 in range(2):
            wait_idx(s0, s0)
            pairs_from_idx(s0)
            issue_gather(s0)

        def body(s, _):
            slot = lax.rem(s, NBUF)

            @pl.when(s + 3 < per_w)
            def _():
                issue_idx(s + 3, lax.rem(s + 3, NBUF))

            @pl.when(s + 2 < per_w)
            def _():
                pslot = lax.rem(s + 2, NBUF)
                wait_idx(s + 2, pslot)
                pairs_from_idx(pslot)

                @pl.when(s + 2 >= NBUF)
                def _():
                    wait_write(pslot)

                issue_gather(pslot)

            wait_gather(slot)

            def col_body(t, _):
                ii_vec = iota + t * _L
                hb = lax.bitwise_and(idx_v[slot, pl.ds(t * _L, _L)], 1) * D
                for c0 in range(0, D, _L):
                    base = hb + c0
                    for d in range(_L):
                        vals = plsc.load_gather(
                            gath_v.at[slot], [ii_vec, base + cdiag[d]]
                        )
                        plsc.store_scatter(
                            slab_v.at[slot], [c0 + cdiag[d], ii_vec], vals
                        )
                return 0

            lax.fori_loop(0, IB // _L, col_body, 0)
            issue_write(s, slot)
            return 0

        lax.fori_loop(0, per_w, body, 0)
        for slot in range(NBUF):
            wait_write(slot)

    return gather_kernel


def kernel(x, table):
    V, D = table.shape
    n, m = x.shape
    tT = table.T
    idxT = x.T
    v_main = (V // _CB) * _CB
    tail = table[v_main:].reshape((V - v_main) // 2, 2 * D)
    scr = _make_transpose(V, D)(tT, tail)
    outT = _make_gather(m, n, D)(idxT, scr)
    return outT.transpose(2, 0, 1)


# speedup vs baseline: 5.3634x; 1.8702x over previous
import functools

import jax
import jax.numpy as jnp
from jax import lax
from jax.experimental import pallas as pl
from jax.experimental.pallas import tpu as pltpu
from jax.experimental.pallas import tpu_sc as plsc

_L = 16
_CB = 384


@functools.lru_cache(maxsize=None)
def _make_transpose(V, D):
    info = plsc.get_sparse_core_info()
    nc, ns = info.num_cores, info.num_subcores
    nw = nc * ns
    n_blocks = (V // _CB)
    v_main = n_blocks * _CB
    tail_rows = (V - v_main) * D // (2 * D)
    mesh = plsc.VectorSubcoreMesh(core_axis_name="c", subcore_axis_name="s")

    @functools.partial(
        pl.kernel,
        mesh=mesh,
        out_type=jax.ShapeDtypeStruct((V // 2, 2 * D), jnp.float32),
        scratch_types=[
            pltpu.VMEM((2, D, _CB + 1), jnp.float32),
            pltpu.VMEM((2, _CB // 2, 2 * D), jnp.float32),
            pltpu.VMEM((tail_rows, 2 * D), jnp.float32),
            pltpu.SemaphoreType.DMA((2,)),
            pltpu.SemaphoreType.DMA((2,)),
        ],
        compiler_params=pltpu.CompilerParams(use_tc_tiling_on_sc=True, needs_layout_passes=False),
    )
    def transpose_kernel(tT_hbm, tail_hbm, scr_hbm, in_v, out_v, tail_v,
                         isem, osem):
        wid = lax.axis_index("s") * nc + lax.axis_index("c")
        nsteps = pl.cdiv(n_blocks - wid, nw)
        iota = lax.iota(jnp.int32, _L)
        cdiag = [lax.bitwise_and(iota + d, _L - 1) for d in range(_L)]

        def issue_read(b, slot):
            pltpu.async_copy(
                tT_hbm.at[:, pl.ds(b * _CB, _CB)],
                in_v.at[slot, :, pl.ds(0, _CB)],
                isem.at[slot],
            )

        issue_read(wid, 0)

        def body(step, _):
            b = wid + nw * step
            slot = lax.rem(step, 2)
            nslot = lax.rem(step + 1, 2)

            @pl.when(step + 1 < nsteps)
            def _():
                issue_read(b + nw, nslot)

            pltpu.make_async_copy(
                tT_hbm.at[:, pl.ds(b * _CB, _CB)],
                in_v.at[slot, :, pl.ds(0, _CB)],
                isem.at[slot],
            ).wait()

            @pl.when(step >= 2)
            def _():
                pltpu.make_async_copy(
                    out_v.at[slot],
                    scr_hbm.at[pl.ds(0, _CB // 2)],
                    osem.at[slot],
                ).wait()

            def row_body(pt, _):
                p_vec = iota + pt * _L
                r_half = [2 * p_vec, 2 * p_vec + 1]
                for q0 in range(0, 2 * D, _L):
                    h = q0 // D
                    srcs = [(q0 - h * D) + cdiag[d] for d in range(_L)]
                    vals = [
                        plsc.load_gather(in_v.at[slot], [srcs[d], r_half[h]])
                        for d in range(_L)
                    ]
                    for d in range(_L):
                        plsc.store_scatter(
                            out_v.at[slot], [p_vec, q0 + cdiag[d]], vals[d]
                        )
                return 0

            lax.fori_loop(0, _CB // 2 // _L, row_body, 0)
            pltpu.async_copy(
                out_v.at[slot],
                scr_hbm.at[pl.ds(b * (_CB // 2), _CB // 2)],
                osem.at[slot],
            )
            return 0

        lax.fori_loop(0, nsteps, body, 0)

        @pl.when(wid == 0)
        def _():
            pltpu.sync_copy(tail_hbm, tail_v)
            pltpu.sync_copy(tail_v, scr_hbm.at[pl.ds(v_main // 2, tail_rows)])

        for slot in range(2):
            pltpu.make_async_copy(
                out_v.at[slot], scr_hbm.at[pl.ds(0, _CB // 2)], osem.at[slot]
            ).wait()

    return transpose_kernel


@functools.lru_cache(maxsize=None)
def _make_gather(J, I, D):
    info = plsc.get_sparse_core_info()
    nc, ns = info.num_cores, info.num_subcores
    nw = nc * ns
    IB = 128
    n_slabs = J * (I // IB)
    per_w = n_slabs // nw
    NBUF = 4
    assert n_slabs % nw == 0 and per_w % NBUF == 0
    mesh = plsc.VectorSubcoreMesh(core_axis_name="c", subcore_axis_name="s")

    @functools.partial(
        pl.kernel,
        mesh=mesh,
        out_type=jax.ShapeDtypeStruct((J, D, I), jnp.float32),
        scratch_types=[
            pltpu.VMEM((NBUF, IB), jnp.int32),
            pltpu.VMEM((NBUF, IB), jnp.int32),
            pltpu.VMEM((NBUF, IB, 2 * D), jnp.float32),
            pltpu.VMEM((NBUF, D, IB), jnp.float32),
            pltpu.SemaphoreType.DMA((NBUF,)),
            pltpu.SemaphoreType.DMA((NBUF,)),
            pltpu.SemaphoreType.DMA((NBUF,)),
        ],
        compiler_params=pltpu.CompilerParams(use_tc_tiling_on_sc=True, needs_layout_passes=False),
    )
    def gather_kernel(idxT_hbm, scr_hbm, out_hbm, idx_v, pair_v, gath_v,
                      slab_v, xsem, gsem, wsem):
        wid = lax.axis_index("s") * nc + lax.axis_index("c")
        iota = lax.iota(jnp.int32, _L)
        cdiag = [lax.bitwise_and(iota + d, _L - 1) for d in range(_L)]

        def slab_ji(s):
            g = wid * per_w + s
            return g // (I // IB), lax.rem(g, I // IB) * IB

        def issue_idx(s, slot):
            j, i0 = slab_ji(s)
            pltpu.async_copy(
                idxT_hbm.at[j, pl.ds(i0, IB)], idx_v.at[slot], xsem.at[slot]
            )

        def wait_idx(s, slot):
            j, i0 = slab_ji(s)
            pltpu.make_async_copy(
                idxT_hbm.at[j, pl.ds(i0, IB)], idx_v.at[slot], xsem.at[slot]
            ).wait()

        def issue_gather(slot):
            pltpu.async_copy(
                scr_hbm.at[pair_v.at[slot]], gath_v.at[slot], gsem.at[slot]
            )

        def wait_gather(slot):
            pltpu.make_async_copy(
                scr_hbm.at[pair_v.at[slot]], gath_v.at[slot], gsem.at[slot]
            ).wait()

        def issue_write(s, slot):
            j, i0 = slab_ji(s)
            pltpu.async_copy(
                slab_v.at[slot], out_hbm.at[j, :, pl.ds(i0, IB)], wsem.at[slot]
            )

        def wait_write(slot):
            pltpu.make_async_copy(
                slab_v.at[slot], out_hbm.at[0, :, pl.ds(0, IB)], wsem.at[slot]
            ).wait()

        def pairs_from_idx(slot):
            for q in range(IB // _L):
                v = idx_v[slot, pl.ds(q * _L, _L)]
                pair_v[slot, pl.ds(q * _L, _L)] = lax.shift_right_logical(v, 1)

        for s0 in range(3):
            issue_idx(s0, s0)
        for s0 in range(2):
            wait_idx(s0, s0)
            pairs_from_idx(s0)
            issue_gather(s0)

        def body(s, _):
            slot = lax.rem(s, NBUF)

            @pl.when(s + 3 < per_w)
            def _():
                issue_idx(s + 3, lax.rem(s + 3, NBUF))

            @pl.when(s + 2 < per_w)
            def _():
                pslot = lax.rem(s + 2, NBUF)
                wait_idx(s + 2, pslot)
                pairs_from_idx(pslot)

                @pl.when(s + 2 >= NBUF)
                def _():
                    wait_write(pslot)

                issue_gather(pslot)

            wait_gather(slot)

            def col_body(t, _):
                ii_vec = iota + t * _L
                hb = lax.bitwise_and(idx_v[slot, pl.ds(t * _L, _L)], 1) * D
                for c0 in range(0, D, _L):
                    base = hb + c0
                    srcs = [base + cdiag[d] for d in range(_L)]
                    vals = [
                        plsc.load_gather(gath_v.at[slot], [ii_vec, srcs[d]])
                        for d in range(_L)
                    ]
                    for d in range(_L):
                        plsc.store_scatter(
                            slab_v.at[slot], [c0 + cdiag[d], ii_vec], vals[d]
                        )
                return 0

            lax.fori_loop(0, IB // _L, col_body, 0)
            issue_write(s, slot)
            return 0

        lax.fori_loop(0, per_w, body, 0)
        for slot in range(NBUF):
            wait_write(slot)

    return gather_kernel


def kernel(x, table):
    V, D = table.shape
    n, m = x.shape
    tT = table.T
    idxT = x.T
    v_main = (V // _CB) * _CB
    tail = table[v_main:].reshape((V - v_main) // 2, 2 * D)
    scr = _make_transpose(V, D)(tT, tail)
    outT = _make_gather(m, n, D)(idxT, scr)
    return outT.transpose(2, 0, 1)
